# baseline (speedup 1.0000x reference)
import functools

import jax
import jax.numpy as jnp
from jax import lax
from jax.experimental import pallas as pl
from jax.experimental.pallas import tpu as pltpu
from jax.experimental.pallas import tpu_sc as plsc

B = 4
T = 2048
H = 1024
NC = 2
NS = 16
NW = NC * NS
NQ = 4
BT = T // NQ
TOKQ = B * BT
PW = TOKQ // NW
CH = 32
NCHUNK = PW // CH
NSLOT = 2
EPS = 1e-12


@functools.partial(
    pl.kernel,
    mesh=plsc.VectorSubcoreMesh(core_axis_name="c", subcore_axis_name="s"),
    out_type=jax.ShapeDtypeStruct((TOKQ, H), jnp.float32),
    scratch_types=[
        pltpu.VMEM((PW,), jnp.int32),
        pltpu.VMEM((NSLOT, CH, H), jnp.float32),
        pltpu.SemaphoreType.DMA,
        pltpu.SemaphoreType.DMA,
        pltpu.SemaphoreType.DMA,
        pltpu.SemaphoreType.DMA,
    ],
)
def _sc_gather(ids_hbm, wemb_hbm, out_hbm, idx_v, rows_v,
               sg0, sg1, so0, so1):
    sg = (sg0, sg1)
    so = (so0, so1)
    c = lax.axis_index("c")
    s = lax.axis_index("s")
    wid = s * NC + c
    base = wid * PW

    pltpu.sync_copy(ids_hbm.at[pl.ds(base, PW)], idx_v)

    def gather_issue(j):
        pltpu.async_copy(wemb_hbm.at[idx_v.at[pl.ds(j * CH, CH)]],
                         rows_v.at[j % NSLOT], sg[j % NSLOT])

    def gather_wait(j):
        pltpu.make_async_copy(wemb_hbm.at[idx_v.at[pl.ds(j * CH, CH)]],
                              rows_v.at[j % NSLOT], sg[j % NSLOT]).wait()

    def out_issue(j):
        pltpu.async_copy(rows_v.at[j % NSLOT],
                         out_hbm.at[pl.ds(base + j * CH, CH)], so[j % NSLOT])

    def out_wait(j):
        pltpu.make_async_copy(rows_v.at[j % NSLOT],
                              out_hbm.at[pl.ds(base + j * CH, CH)],
                              so[j % NSLOT]).wait()

    gather_issue(0)
    gather_issue(1)
    for j in range(NCHUNK):
        if j + 2 < NCHUNK:
            if j >= 1:
                out_wait(j - 1)
            gather_issue(j + 2)
        gather_wait(j)
        out_issue(j)
    out_wait(NCHUNK - 2)
    out_wait(NCHUNK - 1)


def _ln_block(x, g, b):
    mean = jnp.mean(x, axis=-1, keepdims=True)
    xc = x - mean
    var = jnp.mean(xc * xc, axis=-1, keepdims=True)
    return (xc * lax.rsqrt(var + EPS)) * g + b


BTT = 256
NJ = BT // BTT


def _tc_ln0(emb_ref, pos_ref, g_ref, b_ref, o_ref):
    o_ref[0] = _ln_block(emb_ref[0] + pos_ref[0], g_ref[...], b_ref[...])


def _tc_ln1(emb_ref, pos_ref, g_ref, b_ref, buf_ref, o_ref):
    del buf_ref
    o_ref[0] = _ln_block(emb_ref[0] + pos_ref[0], g_ref[...], b_ref[...])


def kernel(input_ids, word_emb, pos_emb, gamma, beta):
    ids = input_ids.astype(jnp.int32)
    gamma2 = gamma.reshape(1, H)
    beta2 = beta.reshape(1, H)
    pos3 = pos_emb.reshape(NQ, BT, H)

    gathered = [
        _sc_gather(ids[:, q * BT:(q + 1) * BT].reshape(-1), word_emb)
        .reshape(B, BT, H)
        for q in range(NQ)
    ]

    def specs(q):
        return [
            pl.BlockSpec((1, BTT, H), lambda j, b: (b, j, 0)),
            pl.BlockSpec((1, BTT, H), lambda j, b, q=q: (q, j, 0)),
            pl.BlockSpec((1, H), lambda j, b: (0, 0)),
            pl.BlockSpec((1, H), lambda j, b: (0, 0)),
        ]

    out_sds = jax.ShapeDtypeStruct((B, T, H), jnp.float32)
    buf = pl.pallas_call(
        _tc_ln0,
        grid=(NJ, B),
        in_specs=specs(0),
        out_specs=pl.BlockSpec((1, BTT, H), lambda j, b: (b, j, 0)),
        out_shape=out_sds,
    )(gathered[0], pos3, gamma2, beta2)
    for q in range(1, NQ):
        buf = pl.pallas_call(
            _tc_ln1,
            grid=(NJ, B),
            in_specs=specs(q) + [pl.BlockSpec((1, BTT, H),
                                              lambda j, b: (0, 0, 0))],
            out_specs=pl.BlockSpec((1, BTT, H),
                                   lambda j, b, q=q: (b, q * NJ + j, 0)),
            out_shape=out_sds,
            input_output_aliases={4: 0},
        )(gathered[q], pos3, gamma2, beta2, buf)
    return buf


# speedup vs baseline: 1.1066x; 1.1066x over previous
import functools

import jax
import jax.numpy as jnp
from jax import lax
from jax.experimental import pallas as pl
from jax.experimental.pallas import tpu as pltpu
from jax.experimental.pallas import tpu_sc as plsc

B = 4
T = 2048
H = 1024
NC = 2
NS = 16
NW = NC * NS
NQ = 2
BT = T // NQ
TOKQ = B * BT
PW = TOKQ // NW
CH = 32
NCHUNK = PW // CH
NSLOT = 3
EPS = 1e-12


@functools.partial(
    pl.kernel,
    mesh=plsc.VectorSubcoreMesh(core_axis_name="c", subcore_axis_name="s"),
    out_type=jax.ShapeDtypeStruct((TOKQ, H), jnp.float32),
    scratch_types=[
        pltpu.VMEM((PW,), jnp.int32),
        pltpu.VMEM((NSLOT, CH, H), jnp.float32),
        pltpu.SemaphoreType.DMA,
        pltpu.SemaphoreType.DMA,
        pltpu.SemaphoreType.DMA,
        pltpu.SemaphoreType.DMA,
        pltpu.SemaphoreType.DMA,
        pltpu.SemaphoreType.DMA,
    ],
)
def _sc_gather(ids_hbm, wemb_hbm, out_hbm, idx_v, rows_v,
               sg0, sg1, sg2, so0, so1, so2):
    sg = (sg0, sg1, sg2)
    so = (so0, so1, so2)
    c = lax.axis_index("c")
    s = lax.axis_index("s")
    wid = s * NC + c
    base = wid * PW

    pltpu.sync_copy(ids_hbm.at[pl.ds(base, PW)], idx_v)

    def gather_issue(j):
        pltpu.async_copy(wemb_hbm.at[idx_v.at[pl.ds(j * CH, CH)]],
                         rows_v.at[j % NSLOT], sg[j % NSLOT])

    def gather_wait(j):
        pltpu.make_async_copy(wemb_hbm.at[idx_v.at[pl.ds(j * CH, CH)]],
                              rows_v.at[j % NSLOT], sg[j % NSLOT]).wait()

    def out_issue(j):
        pltpu.async_copy(rows_v.at[j % NSLOT],
                         out_hbm.at[pl.ds(base + j * CH, CH)], so[j % NSLOT])

    def out_wait(j):
        pltpu.make_async_copy(rows_v.at[j % NSLOT],
                              out_hbm.at[pl.ds(base + j * CH, CH)],
                              so[j % NSLOT]).wait()

    gather_issue(0)
    gather_issue(1)
    for j in range(NCHUNK):
        if j + 2 < NCHUNK:
            if j >= 1:
                out_wait(j - 1)
            gather_issue(j + 2)
        gather_wait(j)
        out_issue(j)
    out_wait(NCHUNK - 2)
    out_wait(NCHUNK - 1)


def _ln_block(x, g, b):
    mean = jnp.mean(x, axis=-1, keepdims=True)
    xc = x - mean
    var = jnp.mean(xc * xc, axis=-1, keepdims=True)
    return (xc * lax.rsqrt(var + EPS)) * g + b


BTT = 512
NJ = BT // BTT


def _tc_ln0(emb_ref, pos_ref, g_ref, b_ref, o_ref):
    o_ref[0] = _ln_block(emb_ref[0] + pos_ref[0], g_ref[...], b_ref[...])


def _tc_ln1(emb_ref, pos_ref, g_ref, b_ref, buf_ref, o_ref):
    del buf_ref
    o_ref[0] = _ln_block(emb_ref[0] + pos_ref[0], g_ref[...], b_ref[...])


def kernel(input_ids, word_emb, pos_emb, gamma, beta):
    ids = input_ids.astype(jnp.int32)
    gamma2 = gamma.reshape(1, H)
    beta2 = beta.reshape(1, H)
    pos3 = pos_emb.reshape(NQ, BT, H)

    gathered = [
        _sc_gather(ids[:, q * BT:(q + 1) * BT].reshape(-1), word_emb)
        .reshape(B, BT, H)
        for q in range(NQ)
    ]

    def specs(q):
        return [
            pl.BlockSpec((1, BTT, H), lambda j, b: (b, j, 0)),
            pl.BlockSpec((1, BTT, H), lambda j, b, q=q: (q, j, 0)),
            pl.BlockSpec((1, H), lambda j, b: (0, 0)),
            pl.BlockSpec((1, H), lambda j, b: (0, 0)),
        ]

    out_sds = jax.ShapeDtypeStruct((B, T, H), jnp.float32)
    buf = pl.pallas_call(
        _tc_ln0,
        grid=(NJ, B),
        in_specs=specs(0),
        out_specs=pl.BlockSpec((1, BTT, H), lambda j, b: (b, j, 0)),
        out_shape=out_sds,
    )(gathered[0], pos3, gamma2, beta2)
    for q in range(1, NQ):
        buf = pl.pallas_call(
            _tc_ln1,
            grid=(NJ, B),
            in_specs=specs(q) + [pl.BlockSpec((1, BTT, H),
                                              lambda j, b: (0, 0, 0))],
            out_specs=pl.BlockSpec((1, BTT, H),
                                   lambda j, b, q=q: (b, q * NJ + j, 0)),
            out_shape=out_sds,
            input_output_aliases={4: 0},
        )(gathered[q], pos3, gamma2, beta2, buf)
    return buf


# speedup vs baseline: 1.1213x; 1.0133x over previous
import functools

import jax
import jax.numpy as jnp
from jax import lax
from jax.experimental import pallas as pl
from jax.experimental.pallas import tpu as pltpu
from jax.experimental.pallas import tpu_sc as plsc

B = 4
T = 2048
H = 1024
NC = 2
NS = 16
NW = NC * NS
NQ = 2
BT = T // NQ
TOKQ = B * BT
PW = TOKQ // NW
CH = 32
NCHUNK = PW // CH
NSLOT = 3
EPS = 1e-12


@functools.partial(
    pl.kernel,
    mesh=plsc.VectorSubcoreMesh(core_axis_name="c", subcore_axis_name="s"),
    out_type=jax.ShapeDtypeStruct((TOKQ, H), jnp.float32),
    scratch_types=[
        pltpu.VMEM((PW,), jnp.int32),
        pltpu.VMEM((NSLOT, CH, H), jnp.float32),
        pltpu.SemaphoreType.DMA,
        pltpu.SemaphoreType.DMA,
        pltpu.SemaphoreType.DMA,
        pltpu.SemaphoreType.DMA,
        pltpu.SemaphoreType.DMA,
        pltpu.SemaphoreType.DMA,
    ],
)
def _sc_gather(ids_hbm, wemb_hbm, out_hbm, idx_v, rows_v,
               sg0, sg1, sg2, so0, so1, so2):
    sg = (sg0, sg1, sg2)
    so = (so0, so1, so2)
    c = lax.axis_index("c")
    s = lax.axis_index("s")
    wid = s * NC + c
    base = wid * PW

    pltpu.sync_copy(ids_hbm.at[pl.ds(base, PW)], idx_v)

    def gather_issue(j):
        pltpu.async_copy(wemb_hbm.at[idx_v.at[pl.ds(j * CH, CH)]],
                         rows_v.at[j % NSLOT], sg[j % NSLOT])

    def gather_wait(j):
        pltpu.make_async_copy(wemb_hbm.at[idx_v.at[pl.ds(j * CH, CH)]],
                              rows_v.at[j % NSLOT], sg[j % NSLOT]).wait()

    def out_issue(j):
        pltpu.async_copy(rows_v.at[j % NSLOT],
                         out_hbm.at[pl.ds(base + j * CH, CH)], so[j % NSLOT])

    def out_wait(j):
        pltpu.make_async_copy(rows_v.at[j % NSLOT],
                              out_hbm.at[pl.ds(base + j * CH, CH)],
                              so[j % NSLOT]).wait()

    gather_issue(0)
    gather_issue(1)
    for j in range(NCHUNK):
        if j + 2 < NCHUNK:
            if j >= 1:
                out_wait(j - 1)
            gather_issue(j + 2)
        gather_wait(j)
        out_issue(j)
    out_wait(NCHUNK - 2)
    out_wait(NCHUNK - 1)


def _ln_block(x, g, b):
    mean = jnp.mean(x, axis=-1, keepdims=True)
    xc = x - mean
    var = jnp.mean(xc * xc, axis=-1, keepdims=True)
    return (xc * lax.rsqrt(var + EPS)) * g + b


BTT = 1024
NJ = BT // BTT


def _tc_ln0(emb_ref, pos_ref, g_ref, b_ref, o_ref):
    o_ref[0] = _ln_block(emb_ref[0] + pos_ref[0], g_ref[...], b_ref[...])


def _tc_ln1(emb_ref, pos_ref, g_ref, b_ref, buf_ref, o_ref):
    del buf_ref
    o_ref[0] = _ln_block(emb_ref[0] + pos_ref[0], g_ref[...], b_ref[...])


def kernel(input_ids, word_emb, pos_emb, gamma, beta):
    ids = input_ids.astype(jnp.int32)
    gamma2 = gamma.reshape(1, H)
    beta2 = beta.reshape(1, H)
    pos3 = pos_emb.reshape(NQ, BT, H)

    gathered = [
        _sc_gather(ids[:, q * BT:(q + 1) * BT].reshape(-1), word_emb)
        .reshape(B, BT, H)
        for q in range(NQ)
    ]

    def specs(q):
        return [
            pl.BlockSpec((1, BTT, H), lambda j, b: (b, j, 0)),
            pl.BlockSpec((1, BTT, H), lambda j, b, q=q: (q, j, 0)),
            pl.BlockSpec((1, H), lambda j, b: (0, 0)),
            pl.BlockSpec((1, H), lambda j, b: (0, 0)),
        ]

    out_sds = jax.ShapeDtypeStruct((B, T, H), jnp.float32)
    buf = pl.pallas_call(
        _tc_ln0,
        grid=(NJ, B),
        in_specs=specs(0),
        out_specs=pl.BlockSpec((1, BTT, H), lambda j, b: (b, j, 0)),
        out_shape=out_sds,
    )(gathered[0], pos3, gamma2, beta2)
    for q in range(1, NQ):
        buf = pl.pallas_call(
            _tc_ln1,
            grid=(NJ, B),
            in_specs=specs(q) + [pl.BlockSpec((1, BTT, H),
                                              lambda j, b: (0, 0, 0))],
            out_specs=pl.BlockSpec((1, BTT, H),
                                   lambda j, b, q=q: (b, q * NJ + j, 0)),
            out_shape=out_sds,
            input_output_aliases={4: 0},
        )(gathered[q], pos3, gamma2, beta2, buf)
    return buf
